# Initial kernel scaffold; baseline (speedup 1.0000x reference)
import jax
import jax.numpy as jnp
from jax.experimental import pallas as pl


def kernel(x, node_weight, edge_index, home, away, W, b):
    raise NotImplementedError("write your pallas kernel here")


# speedup vs baseline: 44.8240x; 44.8240x over previous
import functools

import jax
import jax.numpy as jnp
from jax import lax
from jax.experimental import pallas as pl
from jax.experimental.pallas import tpu as pltpu
from jax.experimental.pallas import tpu_sc as plsc

N = 100000
E = 6400000
B = 4096

NC = 2
NS = 16
NW = NC * NS

GW = 64
ROWS_PER_TILE = E // GW // NW
ROWS_PER_CHUNK = 25
CHUNKS = ROWS_PER_TILE // ROWS_PER_CHUNK
ACC_STRIPE = 6272
ACC_PAD = ACC_STRIPE * NS

_mesh = plsc.VectorSubcoreMesh(core_axis_name="c", subcore_axis_name="s")


@functools.partial(
    pl.kernel,
    out_type=[
        jax.ShapeDtypeStruct((NC, B), jnp.float32),
        jax.ShapeDtypeStruct((NC, B), jnp.float32),
        jax.ShapeDtypeStruct((B,), jnp.float32),
        jax.ShapeDtypeStruct((B,), jnp.float32),
    ],
    mesh=_mesh,
    compiler_params=pltpu.CompilerParams(needs_layout_passes=False),
    scratch_types=[
        pltpu.VMEM((N,), jnp.float32),
        pltpu.VMEM((ROWS_PER_CHUNK, GW), jnp.int32),
        pltpu.VMEM((ROWS_PER_CHUNK, GW), jnp.int32),
        pltpu.VMEM((ROWS_PER_CHUNK, GW), jnp.float32),
        pltpu.VMEM((128,), jnp.int32),
        pltpu.VMEM((128,), jnp.float32),
        pltpu.VMEM((128,), jnp.float32),
        pltpu.VMEM((128,), jnp.float32),
        pltpu.VMEM_SHARED((ACC_PAD,), jnp.float32),
    ],
)
def _sc_pagerank(e3, x_hbm, nw_hbm, home_hbm, away_hbm, zeros_hbm,
                 out_acc_h, out_acc_a, out_xnw_h, out_xnw_a,
                 x_v, rows_v, cols_v, vals_v, idx_v, nw_v, g_v, xnw_v, acc_sh):
    cid = lax.axis_index("c")
    sid = lax.axis_index("s")
    wid = cid * NS + sid

    pltpu.sync_copy(x_hbm, x_v)
    z0 = sid * ACC_STRIPE
    pltpu.sync_copy(zeros_hbm.at[pl.ds(z0, ACC_STRIPE)],
                    acc_sh.at[pl.ds(z0, ACC_STRIPE)])
    plsc.subcore_barrier()

    def chunk_body(c, _):
        pltpu.sync_copy(e3.at[0, wid, c], rows_v)
        pltpu.sync_copy(e3.at[1, wid, c], cols_v)

        def group_body(g, _):
            row_g = rows_v.at[g]
            val_g = vals_v.at[g]
            for k in range(GW // 16):
                idx = row_g[pl.ds(16 * k, 16)]
                val_g[pl.ds(16 * k, 16)] = plsc.load_gather(x_v, [idx])
            pltpu.sync_copy(val_g, acc_sh.at[cols_v.at[g]], add=True)
            return 0

        lax.fori_loop(0, ROWS_PER_CHUNK, group_body, 0)
        return 0

    lax.fori_loop(0, CHUNKS, chunk_body, 0)
    plsc.subcore_barrier()

    for role_hbm, out_acc, out_xnw in (
        (home_hbm, out_acc_h, out_xnw_h),
        (away_hbm, out_acc_a, out_xnw_a),
    ):
        for q in range(2):
            base = sid * 256 + q * 128
            pltpu.sync_copy(role_hbm.at[pl.ds(base, 128)], idx_v)
            pltpu.sync_copy(acc_sh.at[idx_v], g_v)
            pltpu.sync_copy(g_v, out_acc.at[cid, pl.ds(base, 128)])

            @pl.when(cid == 0)
            def _():
                pltpu.sync_copy(nw_hbm.at[idx_v], nw_v)
                for k in range(8):
                    ids = idx_v[pl.ds(16 * k, 16)]
                    xg = plsc.load_gather(x_v, [ids])
                    xnw_v[pl.ds(16 * k, 16)] = xg * nw_v[pl.ds(16 * k, 16)]
                pltpu.sync_copy(xnw_v, out_xnw.at[pl.ds(base, 128)])


def kernel(x, node_weight, edge_index, home, away, W, b):
    x_flat = x.reshape(N)
    nw_flat = node_weight.reshape(N)
    e3 = edge_index.reshape(2, NW, CHUNKS, ROWS_PER_CHUNK, GW)
    zeros = jnp.zeros((ACC_PAD,), jnp.float32)

    acc_h, acc_a, xnw_h, xnw_a = _sc_pagerank(
        e3, x_flat, nw_flat, home, away, zeros)

    w00 = W[0, 0]
    th = xnw_h + (acc_h[0] + acc_h[1]) * w00 + b[0]
    ta = xnw_a + (acc_a[0] + acc_a[1]) * w00 + b[0]
    out = jnp.sign(jax.nn.sigmoid(th) - jax.nn.sigmoid(ta))
    return out.reshape(B, 1)


# speedup vs baseline: 342.8311x; 7.6484x over previous
import functools

import jax
import jax.numpy as jnp
from jax import lax
from jax.experimental import pallas as pl
from jax.experimental.pallas import tpu as pltpu
from jax.experimental.pallas import tpu_sc as plsc

N = 100000
E = 6400000
B = 4096

NC = 2
NS = 16
NW = NC * NS

GW = 128
GROUPS = E // GW
ROWS_PER_CHUNK = 16
TOTAL_CHUNKS = GROUPS // ROWS_PER_CHUNK
ACC_STRIPE = 6272
ACC_PAD = ACC_STRIPE * NS

_mesh = plsc.VectorSubcoreMesh(core_axis_name="c", subcore_axis_name="s")


@functools.partial(
    pl.kernel,
    out_type=[
        jax.ShapeDtypeStruct((NC, B), jnp.float32),
        jax.ShapeDtypeStruct((NC, B), jnp.float32),
        jax.ShapeDtypeStruct((B,), jnp.float32),
        jax.ShapeDtypeStruct((B,), jnp.float32),
    ],
    mesh=_mesh,
    compiler_params=pltpu.CompilerParams(needs_layout_passes=False),
    scratch_types=[
        pltpu.VMEM((N,), jnp.float32),
        pltpu.VMEM((3, ROWS_PER_CHUNK, GW), jnp.int32),
        pltpu.VMEM((3, ROWS_PER_CHUNK, GW), jnp.int32),
        pltpu.VMEM((3, ROWS_PER_CHUNK, GW), jnp.float32),
        pltpu.VMEM((128,), jnp.int32),
        pltpu.VMEM((128,), jnp.float32),
        pltpu.VMEM((128,), jnp.float32),
        pltpu.VMEM((128,), jnp.float32),
        pltpu.VMEM_SHARED((ACC_PAD,), jnp.float32),
        pltpu.SemaphoreType.DMA((3,)),
        pltpu.SemaphoreType.DMA((3,)),
    ],
)
def _sc_pagerank(e3, x_hbm, nw_hbm, home_hbm, away_hbm, zeros_hbm,
                 out_acc_h, out_acc_a, out_xnw_h, out_xnw_a,
                 x_v, rows_v, cols_v, vals_v, idx_v, nw_v, g_v, xnw_v, acc_sh,
                 sem_in, sem_sc):
    cid = lax.axis_index("c")
    sid = lax.axis_index("s")
    wid = cid * NS + sid

    pltpu.sync_copy(x_hbm, x_v)
    z0 = sid * ACC_STRIPE
    pltpu.sync_copy(zeros_hbm.at[pl.ds(z0, ACC_STRIPE)],
                    acc_sh.at[pl.ds(z0, ACC_STRIPE)])
    plsc.subcore_barrier()

    n_chunks = 97 + jnp.where(wid < TOTAL_CHUNKS - 97 * NW, 1, 0)

    def load_chunk(c, slot):
        r0 = (c * NW + wid) * ROWS_PER_CHUNK
        pltpu.async_copy(e3.at[0, pl.ds(r0, ROWS_PER_CHUNK), :],
                         rows_v.at[slot], sem_in.at[slot])
        pltpu.async_copy(e3.at[1, pl.ds(r0, ROWS_PER_CHUNK), :],
                         cols_v.at[slot], sem_in.at[slot])

    def wait_chunk(c, slot):
        r0 = (c * NW + wid) * ROWS_PER_CHUNK
        pltpu.make_async_copy(e3.at[0, pl.ds(r0, ROWS_PER_CHUNK), :],
                              rows_v.at[slot], sem_in.at[slot]).wait()
        pltpu.make_async_copy(e3.at[1, pl.ds(r0, ROWS_PER_CHUNK), :],
                              cols_v.at[slot], sem_in.at[slot]).wait()

    def drain_scatters(slot):
        def body(g, _):
            pltpu.make_async_copy(vals_v.at[slot, g],
                                  acc_sh.at[cols_v.at[slot, g]],
                                  sem_sc.at[slot]).wait()
            return 0
        lax.fori_loop(0, ROWS_PER_CHUNK, body, 0)

    load_chunk(0, 0)

    def chunk_body(c, _):
        m = lax.rem(c, 3)
        wait_chunk(c, m)

        @pl.when(c + 1 < n_chunks)
        def _():
            m1 = lax.rem(c + 1, 3)

            @pl.when(c >= 2)
            def _():
                drain_scatters(m1)
            load_chunk(c + 1, m1)

        def group_body(g, _):
            row_g = rows_v.at[m, g]
            val_g = vals_v.at[m, g]
            for k in range(GW // 16):
                idx = row_g[pl.ds(16 * k, 16)]
                val_g[pl.ds(16 * k, 16)] = plsc.load_gather(x_v, [idx])
            return 0

        lax.fori_loop(0, ROWS_PER_CHUNK, group_body, 0)

        def fire_body(g, _):
            pltpu.async_copy(vals_v.at[m, g], acc_sh.at[cols_v.at[m, g]],
                             sem_sc.at[m], add=True)
            return 0

        lax.fori_loop(0, ROWS_PER_CHUNK, fire_body, 0)
        return 0

    lax.fori_loop(0, n_chunks, chunk_body, 0)
    drain_scatters(lax.rem(n_chunks - 2, 3))
    drain_scatters(lax.rem(n_chunks - 1, 3))
    plsc.subcore_barrier()

    for role_hbm, out_acc, out_xnw in (
        (home_hbm, out_acc_h, out_xnw_h),
        (away_hbm, out_acc_a, out_xnw_a),
    ):
        for q in range(2):
            base = sid * 256 + q * 128
            pltpu.sync_copy(role_hbm.at[pl.ds(base, 128)], idx_v)
            pltpu.sync_copy(acc_sh.at[idx_v], g_v)
            pltpu.sync_copy(g_v, out_acc.at[cid, pl.ds(base, 128)])

            @pl.when(cid == 0)
            def _():
                pltpu.sync_copy(nw_hbm.at[idx_v], nw_v)
                for k in range(8):
                    ids = idx_v[pl.ds(16 * k, 16)]
                    xg = plsc.load_gather(x_v, [ids])
                    xnw_v[pl.ds(16 * k, 16)] = xg * nw_v[pl.ds(16 * k, 16)]
                pltpu.sync_copy(xnw_v, out_xnw.at[pl.ds(base, 128)])


def kernel(x, node_weight, edge_index, home, away, W, b):
    x_flat = x.reshape(N)
    nw_flat = node_weight.reshape(N)
    e3 = edge_index.reshape(2, GROUPS, GW)
    zeros = jnp.zeros((ACC_PAD,), jnp.float32)

    acc_h, acc_a, xnw_h, xnw_a = _sc_pagerank(
        e3, x_flat, nw_flat, home, away, zeros)

    w00 = W[0, 0]
    th = xnw_h + (acc_h[0] + acc_h[1]) * w00 + b[0]
    ta = xnw_a + (acc_a[0] + acc_a[1]) * w00 + b[0]
    out = jnp.sign(jax.nn.sigmoid(th) - jax.nn.sigmoid(ta))
    return out.reshape(B, 1)


# speedup vs baseline: 352.9557x; 1.0295x over previous
import functools

import jax
import jax.numpy as jnp
from jax import lax
from jax.experimental import pallas as pl
from jax.experimental.pallas import tpu as pltpu
from jax.experimental.pallas import tpu_sc as plsc

N = 100000
E = 6400000
B = 4096

NC = 2
NS = 16
NW = NC * NS

GW = 128
GROUPS_PER_CHUNK = 16
CHUNK_E = GW * GROUPS_PER_CHUNK
TOTAL_CHUNKS = E // CHUNK_E
ACC_STRIPE = 6272
ACC_PAD = ACC_STRIPE * NS

_mesh = plsc.VectorSubcoreMesh(core_axis_name="c", subcore_axis_name="s")


@functools.partial(
    pl.kernel,
    out_type=[
        jax.ShapeDtypeStruct((NC, B), jnp.float32),
        jax.ShapeDtypeStruct((NC, B), jnp.float32),
        jax.ShapeDtypeStruct((B,), jnp.float32),
        jax.ShapeDtypeStruct((B,), jnp.float32),
    ],
    mesh=_mesh,
    compiler_params=pltpu.CompilerParams(needs_layout_passes=False),
    scratch_types=[
        pltpu.VMEM((N,), jnp.float32),
        pltpu.VMEM((3 * CHUNK_E,), jnp.int32),
        pltpu.VMEM((3, GROUPS_PER_CHUNK, GW), jnp.int32),
        pltpu.VMEM((3, GROUPS_PER_CHUNK, GW), jnp.float32),
        pltpu.VMEM((128,), jnp.int32),
        pltpu.VMEM((128,), jnp.float32),
        pltpu.VMEM((128,), jnp.float32),
        pltpu.VMEM((128,), jnp.float32),
        pltpu.VMEM_SHARED((ACC_PAD,), jnp.float32),
        pltpu.SemaphoreType.DMA((3,)),
        pltpu.SemaphoreType.DMA((3,)),
    ],
)
def _sc_pagerank(e3, c2, x_hbm, nw_hbm, home_hbm, away_hbm, zeros_hbm,
                 out_acc_h, out_acc_a, out_xnw_h, out_xnw_a,
                 x_v, rows_v, cols_v, vals_v, idx_v, nw_v, g_v, xnw_v, acc_sh,
                 sem_in, sem_sc):
    cid = lax.axis_index("c")
    sid = lax.axis_index("s")
    wid = cid * NS + sid

    pltpu.sync_copy(x_hbm, x_v)
    z0 = sid * ACC_STRIPE
    pltpu.sync_copy(zeros_hbm.at[pl.ds(z0, ACC_STRIPE)],
                    acc_sh.at[pl.ds(z0, ACC_STRIPE)])
    plsc.subcore_barrier()

    n_chunks = 97 + jnp.where(wid < TOTAL_CHUNKS - 97 * NW, 1, 0)

    def load_chunk(c, slot):
        r0 = (c * NW + wid) * CHUNK_E
        g0 = (c * NW + wid) * GROUPS_PER_CHUNK
        pltpu.async_copy(e3.at[0, pl.ds(r0, CHUNK_E)],
                         rows_v.at[pl.ds(slot * CHUNK_E, CHUNK_E)],
                         sem_in.at[slot])
        pltpu.async_copy(c2.at[pl.ds(g0, GROUPS_PER_CHUNK), :],
                         cols_v.at[slot], sem_in.at[slot])

    def wait_chunk(c, slot):
        r0 = (c * NW + wid) * CHUNK_E
        g0 = (c * NW + wid) * GROUPS_PER_CHUNK
        pltpu.make_async_copy(e3.at[0, pl.ds(r0, CHUNK_E)],
                              rows_v.at[pl.ds(slot * CHUNK_E, CHUNK_E)],
                              sem_in.at[slot]).wait()
        pltpu.make_async_copy(c2.at[pl.ds(g0, GROUPS_PER_CHUNK), :],
                              cols_v.at[slot], sem_in.at[slot]).wait()

    def drain_scatters(slot):
        def body(g, _):
            pltpu.make_async_copy(vals_v.at[slot, g],
                                  acc_sh.at[cols_v.at[slot, g]],
                                  sem_sc.at[slot]).wait()
            return 0
        lax.fori_loop(0, GROUPS_PER_CHUNK, body, 0)

    load_chunk(0, 0)

    def chunk_body(c, _):
        m = lax.rem(c, 3)
        wait_chunk(c, m)

        @pl.when(c + 1 < n_chunks)
        def _():
            m1 = lax.rem(c + 1, 3)

            @pl.when(c >= 2)
            def _():
                drain_scatters(m1)
            load_chunk(c + 1, m1)

        def group_body(g, _):
            o = m * CHUNK_E + g * GW
            val_g = vals_v.at[m, g]
            for k in range(GW // 16):
                idx = rows_v[pl.ds(o + 16 * k, 16)]
                val_g[pl.ds(16 * k, 16)] = plsc.load_gather(x_v, [idx])
            return 0

        lax.fori_loop(0, GROUPS_PER_CHUNK, group_body, 0)

        def fire_body(g, _):
            pltpu.async_copy(vals_v.at[m, g], acc_sh.at[cols_v.at[m, g]],
                             sem_sc.at[m], add=True)
            return 0

        lax.fori_loop(0, GROUPS_PER_CHUNK, fire_body, 0)
        return 0

    lax.fori_loop(0, n_chunks, chunk_body, 0)
    drain_scatters(lax.rem(n_chunks - 2, 3))
    drain_scatters(lax.rem(n_chunks - 1, 3))
    plsc.subcore_barrier()

    for role_hbm, out_acc, out_xnw in (
        (home_hbm, out_acc_h, out_xnw_h),
        (away_hbm, out_acc_a, out_xnw_a),
    ):
        for q in range(2):
            base = sid * 256 + q * 128
            pltpu.sync_copy(role_hbm.at[pl.ds(base, 128)], idx_v)
            pltpu.sync_copy(acc_sh.at[idx_v], g_v)
            pltpu.sync_copy(g_v, out_acc.at[cid, pl.ds(base, 128)])

            @pl.when(cid == 0)
            def _():
                pltpu.sync_copy(nw_hbm.at[idx_v], nw_v)
                for k in range(8):
                    ids = idx_v[pl.ds(16 * k, 16)]
                    xg = plsc.load_gather(x_v, [ids])
                    xnw_v[pl.ds(16 * k, 16)] = xg * nw_v[pl.ds(16 * k, 16)]
                pltpu.sync_copy(xnw_v, out_xnw.at[pl.ds(base, 128)])


def kernel(x, node_weight, edge_index, home, away, W, b):
    x_flat = x.reshape(N)
    nw_flat = node_weight.reshape(N)
    e3 = edge_index
    c2 = edge_index[1].reshape(E // GW, GW)
    zeros = jnp.zeros((ACC_PAD,), jnp.float32)

    acc_h, acc_a, xnw_h, xnw_a = _sc_pagerank(
        e3, c2, x_flat, nw_flat, home, away, zeros)

    w00 = W[0, 0]
    th = xnw_h + (acc_h[0] + acc_h[1]) * w00 + b[0]
    ta = xnw_a + (acc_a[0] + acc_a[1]) * w00 + b[0]
    out = jnp.sign(jax.nn.sigmoid(th) - jax.nn.sigmoid(ta))
    return out.reshape(B, 1)
